# baseline (speedup 1.0000x reference)
import functools

import jax
import jax.numpy as jnp
from jax import lax
from jax.experimental import pallas as pl
from jax.experimental.pallas import tpu as pltpu
from jax.experimental.pallas import tpu_sc as plsc

_CHUNK = 128


def kernel(class_id, table):
    (B,) = class_id.shape
    V, D = table.shape
    pack = 128 // D
    table128 = table.reshape(V // pack, 128)

    info = plsc.get_sparse_core_info()
    NC, NS = info.num_cores, info.num_subcores
    NW = NC * NS
    b_per_w = B // NW
    n_chunks = b_per_w // _CHUNK
    n_groups = b_per_w // 16

    idx = class_id.astype(jnp.int32)
    q3 = (idx // pack).reshape(NW, n_chunks, _CHUNK)
    r2 = (idx % pack).reshape(NW, b_per_w)

    mesh = plsc.VectorSubcoreMesh(core_axis_name="c", subcore_axis_name="s")

    @functools.partial(
        pl.kernel,
        mesh=mesh,
        out_type=jax.ShapeDtypeStruct((B // pack, 128), jnp.float32),
        compiler_params=pltpu.CompilerParams(needs_layout_passes=False),
        scratch_types=[
            pltpu.VMEM((n_chunks, _CHUNK), jnp.int32),
            pltpu.VMEM((b_per_w,), jnp.int32),
            pltpu.VMEM((b_per_w, 128), jnp.float32),
            pltpu.VMEM((b_per_w // pack, 128), jnp.float32),
            pltpu.SemaphoreType.DMA,
        ],
    )
    def emb(table_hbm, q_hbm, r_hbm, out_hbm, q_v, r_v, quad_v, comp_v, sem):
        wid = lax.axis_index("s") * NC + lax.axis_index("c")
        pltpu.sync_copy(q_hbm.at[wid], q_v)
        pltpu.sync_copy(r_hbm.at[wid], r_v)
        copies = [
            pltpu.async_copy(
                table_hbm.at[q_v.at[j]],
                quad_v.at[pl.ds(j * _CHUNK, _CHUNK)],
                sem,
            )
            for j in range(n_chunks)
        ]
        for c in copies:
            c.wait()

        lane = lax.iota(jnp.int32, 16)

        def body(g, carry):
            rows16 = g * 16 + lane
            rv = plsc.load_gather(r_v, [rows16])
            colbase = rv * D
            orow = rows16 // pack
            ocolbase = (rows16 % pack) * D
            for c in range(D):
                vals = plsc.load_gather(quad_v, [rows16, colbase + c])
                plsc.store_scatter(comp_v, [orow, ocolbase + c], vals)
            return carry

        lax.fori_loop(0, n_groups, body, 0)
        pltpu.sync_copy(
            comp_v, out_hbm.at[pl.ds(wid * (b_per_w // pack), b_per_w // pack)]
        )

    out128 = emb(table128, q3, r2)
    return out128.reshape(B, D)


# speedup vs baseline: 1.7202x; 1.7202x over previous
import functools

import jax
import jax.numpy as jnp
from jax import lax
from jax.experimental import pallas as pl
from jax.experimental.pallas import tpu as pltpu
from jax.experimental.pallas import tpu_sc as plsc


def kernel(class_id, table):
    (B,) = class_id.shape
    V, D = table.shape
    info = plsc.get_sparse_core_info()
    NC, NS = info.num_cores, info.num_subcores
    NW = NC * NS
    b_per_w = B // NW

    q2 = class_id.astype(jnp.int32).reshape(NW, b_per_w)
    mesh = plsc.VectorSubcoreMesh(core_axis_name="c", subcore_axis_name="s")

    @functools.partial(
        pl.kernel,
        mesh=mesh,
        out_type=jax.ShapeDtypeStruct((B, D), jnp.float32),
        scratch_types=[
            pltpu.VMEM((b_per_w,), jnp.int32),
            pltpu.VMEM((b_per_w, D), jnp.float32),
            pltpu.SemaphoreType.DMA,
        ],
    )
    def emb(table_hbm, q_hbm, out_hbm, q_v, rows_v, sem):
        wid = lax.axis_index("s") * NC + lax.axis_index("c")
        pltpu.sync_copy(q_hbm.at[wid], q_v)

        def issue(g, carry):
            vec = q_v[pl.ds(g * 16, 16)]
            for k in range(16):
                pltpu.make_async_copy(
                    table_hbm.at[pl.ds(vec[k], 1)],
                    rows_v.at[pl.ds(g * 16 + k, 1)],
                    sem,
                ).start()
            return carry

        lax.fori_loop(0, b_per_w // 16, issue, 0)

        def drain(i, carry):
            pltpu.make_async_copy(
                table_hbm.at[pl.ds(0, 1)], rows_v.at[pl.ds(i, 1)], sem
            ).wait()
            return carry

        lax.fori_loop(0, b_per_w, drain, 0)
        pltpu.sync_copy(rows_v, out_hbm.at[pl.ds(wid * b_per_w, b_per_w)])

    return emb(table, q2)
